# Initial kernel scaffold; baseline (speedup 1.0000x reference)
import jax
import jax.numpy as jnp
from jax.experimental import pallas as pl


def kernel(x, edge_index, Wl1, bl1, Wr1, gamma, beta, Wl2, bl2, Wr2):
    raise NotImplementedError("write your pallas kernel here")


# speedup vs baseline: 6.9921x; 6.9921x over previous
import dataclasses
import functools

import jax
import jax.numpy as jnp
from jax import lax
from jax.experimental import pallas as pl
from jax.experimental.pallas import tpu as pltpu
from jax.experimental.pallas import tpu_sc as plsc

NC = 2
NS = 16
NW = NC * NS
CH = 128
CW = 16


def _chunks(total, step=CH):
    out, o = [], 0
    while o < total:
        sz = min(step, total - o)
        out.append((o, sz))
        o += sz
    return out


def _make_seg_sum(n, d, e, with_cnt):
    epw = e // NW
    nfull = epw // CH
    rem = epw - nfull * CH
    assert epw * NW == e and rem % 16 == 0
    rpt_a = -(-(n // NS) // 8) * 8
    rpt_b = n - (NS - 1) * rpt_a
    assert 0 < rpt_b <= rpt_a

    mesh = plsc.VectorSubcoreMesh(core_axis_name="c", subcore_axis_name="s")
    out_type = [jax.ShapeDtypeStruct((NC, n, d), jnp.float32)]
    scratch = [
        pltpu.VMEM((CH,), jnp.int32),
        pltpu.VMEM((CH,), jnp.int32),
        pltpu.VMEM((CH, d), jnp.float32),
        pltpu.VMEM_SHARED((n, d), jnp.float32),
    ]
    if with_cnt:
        out_type.append(jax.ShapeDtypeStruct((NW, 1, n), jnp.float32))
        scratch.append(pltpu.VMEM((1, n), jnp.float32))
    if rem:
        scratch += [
            pltpu.VMEM((rem,), jnp.int32),
            pltpu.VMEM((rem,), jnp.int32),
            pltpu.VMEM((rem, d), jnp.float32),
        ]

    cp = pltpu.CompilerParams()
    if "needs_layout_passes" in pltpu.CompilerParams.__dataclass_fields__:
        cp = dataclasses.replace(cp, needs_layout_passes=False)

    @functools.partial(pl.kernel, mesh=mesh, out_type=out_type,
                       scratch_types=scratch, compiler_params=cp)
    def seg_sum(x_hbm, src_hbm, dst_hbm, agg_hbm, *rest):
        rest = list(rest)
        cnt_hbm = rest.pop(0) if with_cnt else None
        sidx, didx, rows, agg_sp = rest[:4]
        rest = rest[4:]
        if with_cnt:
            cnt_v = rest.pop(0)
        if rem:
            sidx_r, didx_r, rows_r = rest

        c = lax.axis_index("c")
        s = lax.axis_index("s")
        w = c * NS + s
        z16 = jnp.zeros((16,), jnp.float32)
        z16i = jnp.zeros((16,), jnp.int32)
        one16 = jnp.full((16,), 1.0, jnp.float32)

        @pl.loop(0, CH)
        def _(r):
            for cc in range(d // 16):
                rows[r, pl.ds(cc * 16, 16)] = z16

        if with_cnt:
            @pl.loop(0, n // 16)
            def _(i):
                cnt_v[0, pl.ds(i * 16, 16)] = z16

        def zero_fill(base, size):
            for (o, sz) in _chunks(size):
                pltpu.sync_copy(rows.at[pl.ds(0, sz)],
                                agg_sp.at[pl.ds(base + o, sz)])

        def flush(base, size):
            pltpu.sync_copy(agg_sp.at[pl.ds(base, size)],
                            agg_hbm.at[c].at[pl.ds(base, size)])

        base_a = pl.multiple_of(s * rpt_a, 8)

        @pl.when(s < NS - 1)
        def _():
            zero_fill(base_a, rpt_a)

        @pl.when(s == NS - 1)
        def _():
            zero_fill((NS - 1) * rpt_a, rpt_b)

        plsc.subcore_barrier()

        ebase = w * epw

        @pl.loop(0, nfull)
        def _(i):
            off = ebase + i * CH
            pltpu.sync_copy(src_hbm.at[pl.ds(off, CH)], sidx)
            pltpu.sync_copy(dst_hbm.at[pl.ds(off, CH)], didx)
            pltpu.sync_copy(x_hbm.at[sidx], rows)
            pltpu.sync_copy(rows, agg_sp.at[didx], add=True)
            if with_cnt:
                for kk in range(CH // 16):
                    idx16 = didx[pl.ds(kk * 16, 16)]
                    plsc.addupdate_scatter(cnt_v, [z16i, idx16], one16)

        if rem:
            off = ebase + nfull * CH
            pltpu.sync_copy(src_hbm.at[pl.ds(off, rem)], sidx_r)
            pltpu.sync_copy(dst_hbm.at[pl.ds(off, rem)], didx_r)
            pltpu.sync_copy(x_hbm.at[sidx_r], rows_r)
            pltpu.sync_copy(rows_r, agg_sp.at[didx_r], add=True)
            if with_cnt:
                for kk in range(rem // 16):
                    idx16 = didx_r[pl.ds(kk * 16, 16)]
                    plsc.addupdate_scatter(cnt_v, [z16i, idx16], one16)

        plsc.subcore_barrier()

        @pl.when(s < NS - 1)
        def _():
            flush(base_a, rpt_a)

        @pl.when(s == NS - 1)
        def _():
            flush((NS - 1) * rpt_a, rpt_b)

        if with_cnt:
            pltpu.sync_copy(cnt_v, cnt_hbm.at[w])

    return seg_sum


def _dot_t(a, w):
    return lax.dot_general(a, w, (((1,), (1,)), ((), ())),
                           preferred_element_type=jnp.float32)


def _tc1_body(aggp, cntp, x, wl, bl, wr, gamma, beta, h_out, invc_out):
    cnt = jnp.sum(cntp[...], axis=1, keepdims=True)
    invc = 1.0 / jnp.maximum(cnt, 1.0)
    mean_agg = (aggp[0] + aggp[1]) * invc
    h = _dot_t(mean_agg, wl[...]) + bl[...][None, :] + _dot_t(x[...], wr[...])
    mu = jnp.mean(h, axis=0, keepdims=True)
    hc = h - mu
    var = jnp.mean(hc * hc, axis=0, keepdims=True)
    hn = hc / jnp.sqrt(var + 1e-5) * gamma[...][None, :] + beta[...][None, :]
    h_out[...] = jnp.maximum(hn, 0.0)
    invc_out[...] = invc


def _tc2_body(aggp, invc, h, wl, bl, wr, out):
    mean_agg = (aggp[0] + aggp[1]) * invc[...]
    out[...] = (_dot_t(mean_agg, wl[...]) + bl[...][None, :]
                + _dot_t(h[...], wr[...]))


def kernel(x, edge_index, Wl1, bl1, Wr1, gamma, beta, Wl2, bl2, Wr2):
    n, d = x.shape
    e = edge_index.shape[1]
    src = edge_index[0].astype(jnp.int32)
    dst = edge_index[1].astype(jnp.int32)

    seg_sum_cnt = _make_seg_sum(n, d, e, with_cnt=True)
    seg_sum = _make_seg_sum(n, d, e, with_cnt=False)

    agg1, cntp = seg_sum_cnt(x, src, dst)
    cnt_t = cntp.reshape(NW, n).T

    h, invc = pl.pallas_call(
        _tc1_body,
        out_shape=[jax.ShapeDtypeStruct((n, d), jnp.float32),
                   jax.ShapeDtypeStruct((n, 1), jnp.float32)],
    )(agg1, cnt_t, x, Wl1, bl1, Wr1, gamma, beta)

    (agg2,) = seg_sum(h, src, dst)

    out = pl.pallas_call(
        _tc2_body,
        out_shape=jax.ShapeDtypeStruct((n, d), jnp.float32),
    )(agg2, invc, h, Wl2, bl2, Wr2)
    return out


# speedup vs baseline: 9.5045x; 1.3593x over previous
import dataclasses
import functools

import jax
import jax.numpy as jnp
from jax import lax
from jax.experimental import pallas as pl
from jax.experimental.pallas import tpu as pltpu
from jax.experimental.pallas import tpu_sc as plsc

NC = 2
NS = 16
NW = NC * NS
CH = 64
CW = 16


def _chunks(total, step=CH):
    out, o = [], 0
    while o < total:
        sz = min(step, total - o)
        out.append((o, sz))
        o += sz
    return out


def _make_seg_sum(n, d, e, with_cnt):
    epw = e // NW
    nfull = epw // CH
    rem = epw - nfull * CH
    assert epw * NW == e and rem % 16 == 0
    rpt_a = -(-(n // NS) // 8) * 8
    rpt_b = n - (NS - 1) * rpt_a
    assert 0 < rpt_b <= rpt_a

    assert nfull >= 2 and nfull % 2 == 0
    mesh = plsc.VectorSubcoreMesh(core_axis_name="c", subcore_axis_name="s")
    out_type = [jax.ShapeDtypeStruct((NC, n, d), jnp.float32)]
    scratch = [
        pltpu.VMEM((epw,), jnp.int32),
        pltpu.VMEM((epw,), jnp.int32),
        pltpu.VMEM((CH,), jnp.int32),
        pltpu.VMEM((CH,), jnp.int32),
        pltpu.VMEM((CH, d), jnp.float32),
        pltpu.VMEM((CH, d), jnp.float32),
        pltpu.VMEM_SHARED((n, d), jnp.float32),
        pltpu.SemaphoreType.DMA,
        pltpu.SemaphoreType.DMA,
        pltpu.SemaphoreType.DMA,
        pltpu.SemaphoreType.DMA,
    ]
    if with_cnt:
        out_type.append(jax.ShapeDtypeStruct((NW, 1, n), jnp.float32))
        scratch.append(pltpu.VMEM((1, n), jnp.float32))
    if rem:
        scratch.append(pltpu.VMEM((rem,), jnp.int32))

    cp = pltpu.CompilerParams()
    if "needs_layout_passes" in pltpu.CompilerParams.__dataclass_fields__:
        cp = dataclasses.replace(cp, needs_layout_passes=False)

    @functools.partial(pl.kernel, mesh=mesh, out_type=out_type,
                       scratch_types=scratch, compiler_params=cp)
    def seg_sum(x_hbm, src_hbm, dst_hbm, agg_hbm, *rest):
        rest = list(rest)
        cnt_hbm = rest.pop(0) if with_cnt else None
        (sidx_all, didx_all, didx0, didx1, rows0, rows1, agg_sp,
         gsem0, gsem1, isem0, isem1) = rest[:11]
        rest = rest[11:]
        if with_cnt:
            cnt_v = rest.pop(0)
        if rem:
            (didx_r,) = rest

        didx_b = (didx0, didx1)
        rows_b = (rows0, rows1)
        gsem_b = (gsem0, gsem1)

        c = lax.axis_index("c")
        s = lax.axis_index("s")
        w = c * NS + s
        z16 = jnp.zeros((16,), jnp.float32)
        z16i = jnp.zeros((16,), jnp.int32)
        one16 = jnp.full((16,), 1.0, jnp.float32)
        ebase = w * epw

        pltpu.async_copy(src_hbm.at[pl.ds(ebase, epw)], sidx_all, isem0)
        pltpu.async_copy(dst_hbm.at[pl.ds(ebase, epw)], didx_all, isem1)

        @pl.loop(0, CH)
        def _(r):
            for cc in range(d // 16):
                rows0[r, pl.ds(cc * 16, 16)] = z16

        if with_cnt:
            @pl.loop(0, n // 16)
            def _(i):
                cnt_v[0, pl.ds(i * 16, 16)] = z16

        def zero_fill(base, size):
            for (o, sz) in _chunks(size):
                pltpu.sync_copy(rows0.at[pl.ds(0, sz)],
                                agg_sp.at[pl.ds(base + o, sz)])

        def flush(base, size):
            pltpu.sync_copy(agg_sp.at[pl.ds(base, size)],
                            agg_hbm.at[c].at[pl.ds(base, size)])

        base_a = pl.multiple_of(s * rpt_a, 8)

        @pl.when(s < NS - 1)
        def _():
            zero_fill(base_a, rpt_a)

        @pl.when(s == NS - 1)
        def _():
            zero_fill((NS - 1) * rpt_a, rpt_b)

        pltpu.make_async_copy(src_hbm.at[pl.ds(0, epw)], sidx_all, isem0).wait()
        pltpu.make_async_copy(dst_hbm.at[pl.ds(0, epw)], didx_all, isem1).wait()
        plsc.subcore_barrier()

        def g_start(i, slot):
            pltpu.async_copy(x_hbm.at[sidx_all.at[pl.ds(i * CH, CH)]],
                             rows_b[slot], gsem_b[slot])

        def g_wait(slot):
            pltpu.make_async_copy(x_hbm.at[pl.ds(0, CH)], rows_b[slot],
                                  gsem_b[slot]).wait()

        def do_chunk(i, slot):
            for kk in range(CH // 16):
                didx_b[slot][pl.ds(kk * 16, 16)] = (
                    didx_all[pl.ds(i * CH + kk * 16, 16)])
            pltpu.sync_copy(rows_b[slot], agg_sp.at[didx_b[slot]], add=True)
            if with_cnt:
                for kk in range(CH // 16):
                    idx16 = didx_all[pl.ds(i * CH + kk * 16, 16)]
                    plsc.addupdate_scatter(cnt_v, [z16i, idx16], one16)

        g_start(0, 0)

        @pl.loop(0, nfull // 2 - 1)
        def _(j):
            i0 = 2 * j
            g_wait(0)
            g_start(i0 + 1, 1)
            do_chunk(i0, 0)
            g_wait(1)
            g_start(i0 + 2, 0)
            do_chunk(i0 + 1, 1)

        g_wait(0)
        g_start(nfull - 1, 1)
        do_chunk(nfull - 2, 0)
        g_wait(1)
        do_chunk(nfull - 1, 1)

        if rem:
            off = nfull * CH
            for kk in range(rem // 16):
                didx_r[pl.ds(kk * 16, 16)] = (
                    didx_all[pl.ds(off + kk * 16, 16)])
            pltpu.sync_copy(x_hbm.at[sidx_all.at[pl.ds(off, rem)]],
                            rows0.at[pl.ds(0, rem)])
            pltpu.sync_copy(rows0.at[pl.ds(0, rem)],
                            agg_sp.at[didx_r], add=True)
            if with_cnt:
                for kk in range(rem // 16):
                    idx16 = didx_all[pl.ds(off + kk * 16, 16)]
                    plsc.addupdate_scatter(cnt_v, [z16i, idx16], one16)

        plsc.subcore_barrier()

        @pl.when(s < NS - 1)
        def _():
            flush(base_a, rpt_a)

        @pl.when(s == NS - 1)
        def _():
            flush((NS - 1) * rpt_a, rpt_b)

        if with_cnt:
            pltpu.sync_copy(cnt_v, cnt_hbm.at[w])

    return seg_sum


def _dot_t(a, w):
    return lax.dot_general(a, w, (((1,), (1,)), ((), ())),
                           preferred_element_type=jnp.float32)


def _tc1_body(aggp, cntp, x, wl, bl, wr, gamma, beta, h_out, invc_out):
    cnt = jnp.sum(cntp[...], axis=1, keepdims=True)
    invc = 1.0 / jnp.maximum(cnt, 1.0)
    mean_agg = (aggp[0] + aggp[1]) * invc
    h = _dot_t(mean_agg, wl[...]) + bl[...][None, :] + _dot_t(x[...], wr[...])
    mu = jnp.mean(h, axis=0, keepdims=True)
    hc = h - mu
    var = jnp.mean(hc * hc, axis=0, keepdims=True)
    hn = hc / jnp.sqrt(var + 1e-5) * gamma[...][None, :] + beta[...][None, :]
    h_out[...] = jnp.maximum(hn, 0.0)
    invc_out[...] = invc


def _tc2_body(aggp, invc, h, wl, bl, wr, out):
    mean_agg = (aggp[0] + aggp[1]) * invc[...]
    out[...] = (_dot_t(mean_agg, wl[...]) + bl[...][None, :]
                + _dot_t(h[...], wr[...]))


def kernel(x, edge_index, Wl1, bl1, Wr1, gamma, beta, Wl2, bl2, Wr2):
    n, d = x.shape
    e = edge_index.shape[1]
    src = edge_index[0].astype(jnp.int32)
    dst = edge_index[1].astype(jnp.int32)

    seg_sum_cnt = _make_seg_sum(n, d, e, with_cnt=True)
    seg_sum = _make_seg_sum(n, d, e, with_cnt=False)

    agg1, cntp = seg_sum_cnt(x, src, dst)
    cnt_t = cntp.reshape(NW, n).T

    h, invc = pl.pallas_call(
        _tc1_body,
        out_shape=[jax.ShapeDtypeStruct((n, d), jnp.float32),
                   jax.ShapeDtypeStruct((n, 1), jnp.float32)],
    )(agg1, cnt_t, x, Wl1, bl1, Wr1, gamma, beta)

    (agg2,) = seg_sum(h, src, dst)

    out = pl.pallas_call(
        _tc2_body,
        out_shape=jax.ShapeDtypeStruct((n, d), jnp.float32),
    )(agg2, invc, h, Wl2, bl2, Wr2)
    return out


# speedup vs baseline: 11.7632x; 1.2376x over previous
import dataclasses
import functools

import jax
import jax.numpy as jnp
from jax import lax
from jax.experimental import pallas as pl
from jax.experimental.pallas import tpu as pltpu
from jax.experimental.pallas import tpu_sc as plsc

NC = 2
NS = 16
NW = NC * NS
CH = 128


def _chunks(total, step=CH):
    out, o = [], 0
    while o < total:
        sz = min(step, total - o)
        out.append((o, sz))
        o += sz
    return out


def _make_seg_sum(n, d, e, with_cnt):
    epw = e // NW
    nfull = epw // CH
    rem = epw - nfull * CH
    assert epw * NW == e and rem % 16 == 0 and rem < CH
    assert nfull >= 4 and nfull % 2 == 0
    rpt_a = -(-(n // NS) // 8) * 8
    rpt_b = n - (NS - 1) * rpt_a
    assert 0 < rpt_b <= rpt_a

    mesh = plsc.VectorSubcoreMesh(core_axis_name="c", subcore_axis_name="s")
    out_type = [jax.ShapeDtypeStruct((NC, n, d), jnp.float32)]
    scratch = [
        pltpu.VMEM((2, CH), jnp.int32),
        pltpu.VMEM((2, CH), jnp.int32),
        pltpu.VMEM((CH,), jnp.int32),
        pltpu.VMEM((CH,), jnp.int32),
        pltpu.VMEM((CH, d), jnp.float32),
        pltpu.VMEM((CH, d), jnp.float32),
        pltpu.VMEM_SHARED((n, d), jnp.float32),
        pltpu.SemaphoreType.DMA,
        pltpu.SemaphoreType.DMA,
        pltpu.SemaphoreType.DMA,
        pltpu.SemaphoreType.DMA,
        pltpu.SemaphoreType.DMA,
        pltpu.SemaphoreType.DMA,
    ]
    if with_cnt:
        out_type.append(jax.ShapeDtypeStruct((NW, 1, n), jnp.float32))
        scratch.append(pltpu.VMEM((1, n), jnp.float32))
    if rem:
        scratch.append(pltpu.VMEM((2, rem), jnp.int32))

    cp = pltpu.CompilerParams()
    if "needs_layout_passes" in pltpu.CompilerParams.__dataclass_fields__:
        cp = dataclasses.replace(cp, needs_layout_passes=False)

    @functools.partial(pl.kernel, mesh=mesh, out_type=out_type,
                       scratch_types=scratch, compiler_params=cp)
    def seg_sum(x_hbm, idx_hbm, rem_hbm, agg_hbm, *rest):
        rest = list(rest)
        cnt_hbm = rest.pop(0) if with_cnt else None
        (ib0, ib1, didx0, didx1, rows0, rows1, agg_sp,
         isem0, isem1, gsem0, gsem1, ssem0, ssem1) = rest[:13]
        rest = rest[13:]
        if with_cnt:
            cnt_v = rest.pop(0)
        if rem:
            (rbuf,) = rest

        ibufs = (ib0, ib1)
        isems = (isem0, isem1)
        didx_b = (didx0, didx1)
        rows_b = (rows0, rows1)
        gsems = (gsem0, gsem1)
        ssems = (ssem0, ssem1)

        c = lax.axis_index("c")
        s = lax.axis_index("s")
        w = c * NS + s
        z16 = jnp.zeros((16,), jnp.float32)
        z16i = jnp.zeros((16,), jnp.int32)
        one16 = jnp.full((16,), 1.0, jnp.float32)

        def i_start(i, slot):
            pltpu.async_copy(idx_hbm.at[w, i], ibufs[slot], isems[slot])

        def i_wait(i, slot):
            pltpu.make_async_copy(idx_hbm.at[w, i], ibufs[slot],
                                  isems[slot]).wait()

        def g_start(slot):
            pltpu.async_copy(x_hbm.at[ibufs[slot].at[0]], rows_b[slot],
                             gsems[slot])

        def g_wait(slot):
            pltpu.make_async_copy(x_hbm.at[ibufs[slot].at[0]], rows_b[slot],
                                  gsems[slot]).wait()

        def didx_copy(slot):
            for kk in range(CH // 16):
                didx_b[slot][pl.ds(kk * 16, 16)] = (
                    ibufs[slot][1, pl.ds(kk * 16, 16)])

        def s_start(slot):
            pltpu.async_copy(rows_b[slot], agg_sp.at[didx_b[slot]],
                             ssems[slot], add=True)

        def s_wait(slot):
            pltpu.make_async_copy(rows_b[slot], agg_sp.at[didx_b[slot]],
                                  ssems[slot]).wait()

        def cnt_upd(slot):
            if with_cnt:
                for kk in range(CH // 16):
                    idx16 = didx_b[slot][pl.ds(kk * 16, 16)]
                    plsc.addupdate_scatter(cnt_v, [z16i, idx16], one16)

        i_start(0, 0)
        i_start(1, 1)

        @pl.loop(0, CH)
        def _(r):
            for cc in range(d // 16):
                rows0[r, pl.ds(cc * 16, 16)] = z16

        if with_cnt:
            @pl.loop(0, n // 16)
            def _(i):
                cnt_v[0, pl.ds(i * 16, 16)] = z16

        def zero_fill(base, size):
            for (o, sz) in _chunks(size):
                pltpu.sync_copy(rows0.at[pl.ds(0, sz)],
                                agg_sp.at[pl.ds(base + o, sz)])

        def flush(base, size):
            pltpu.sync_copy(agg_sp.at[pl.ds(base, size)],
                            agg_hbm.at[c].at[pl.ds(base, size)])

        base_a = pl.multiple_of(s * rpt_a, 8)

        @pl.when(s < NS - 1)
        def _():
            zero_fill(base_a, rpt_a)

        @pl.when(s == NS - 1)
        def _():
            zero_fill((NS - 1) * rpt_a, rpt_b)

        plsc.subcore_barrier()

        def body(i, slot, start_idx, start_gather, first=False):
            g_wait(slot)
            didx_copy(slot)
            if start_idx:
                i_start(i + 2, slot)
            if not first:
                s_wait(1 - slot)
            s_start(slot)
            cnt_upd(slot)
            if start_gather:
                i_wait(i + 1, 1 - slot)
                g_start(1 - slot)

        i_wait(0, 0)
        g_start(0)
        body(0, 0, True, True, first=True)

        @pl.loop(0, (nfull - 4) // 2)
        def _(j):
            i0 = 2 * j + 1
            body(i0, 1, True, True)
            body(i0 + 1, 0, True, True)

        body(nfull - 3, 1, True, True)
        body(nfull - 2, 0, False, True)
        body(nfull - 1, 1, False, False)
        s_wait(1)

        if rem:
            pltpu.sync_copy(rem_hbm.at[w], rbuf)
            pltpu.sync_copy(x_hbm.at[rbuf.at[0]], rows0.at[pl.ds(0, rem)])
            pltpu.sync_copy(rows0.at[pl.ds(0, rem)],
                            agg_sp.at[rbuf.at[1]], add=True)
            if with_cnt:
                for kk in range(rem // 16):
                    idx16 = rbuf[1, pl.ds(kk * 16, 16)]
                    plsc.addupdate_scatter(cnt_v, [z16i, idx16], one16)

        plsc.subcore_barrier()

        @pl.when(s < NS - 1)
        def _():
            flush(base_a, rpt_a)

        @pl.when(s == NS - 1)
        def _():
            flush((NS - 1) * rpt_a, rpt_b)

        if with_cnt:
            pltpu.sync_copy(cnt_v, cnt_hbm.at[w])

    return seg_sum


def _dot_t(a, w):
    return lax.dot_general(a, w, (((1,), (1,)), ((), ())),
                           preferred_element_type=jnp.float32)


def _tc1_body(aggp, cntp, x, wl, bl, wr, gamma, beta, h_out, invc_out):
    cnt = jnp.sum(cntp[...], axis=1, keepdims=True)
    invc = 1.0 / jnp.maximum(cnt, 1.0)
    mean_agg = (aggp[0] + aggp[1]) * invc
    h = _dot_t(mean_agg, wl[...]) + bl[...][None, :] + _dot_t(x[...], wr[...])
    mu = jnp.mean(h, axis=0, keepdims=True)
    hc = h - mu
    var = jnp.mean(hc * hc, axis=0, keepdims=True)
    hn = hc / jnp.sqrt(var + 1e-5) * gamma[...][None, :] + beta[...][None, :]
    h_out[...] = jnp.maximum(hn, 0.0)
    invc_out[...] = invc


def _tc2_body(aggp, invc, h, wl, bl, wr, out):
    mean_agg = (aggp[0] + aggp[1]) * invc[...]
    out[...] = (_dot_t(mean_agg, wl[...]) + bl[...][None, :]
                + _dot_t(h[...], wr[...]))


def kernel(x, edge_index, Wl1, bl1, Wr1, gamma, beta, Wl2, bl2, Wr2):
    n, d = x.shape
    e = edge_index.shape[1]
    src = edge_index[0].astype(jnp.int32)
    dst = edge_index[1].astype(jnp.int32)

    epw = e // NW
    nfull = epw // CH
    rem = epw - nfull * CH
    srcw = src.reshape(NW, epw)
    dstw = dst.reshape(NW, epw)
    idx_blocks = jnp.stack(
        [srcw[:, :nfull * CH].reshape(NW, nfull, CH),
         dstw[:, :nfull * CH].reshape(NW, nfull, CH)], axis=2)
    rem_blocks = jnp.stack([srcw[:, nfull * CH:], dstw[:, nfull * CH:]],
                           axis=1)

    seg_sum_cnt = _make_seg_sum(n, d, e, with_cnt=True)
    seg_sum = _make_seg_sum(n, d, e, with_cnt=False)

    agg1, cntp = seg_sum_cnt(x, idx_blocks, rem_blocks)
    cnt_t = cntp.reshape(NW, n).T

    h, invc = pl.pallas_call(
        _tc1_body,
        out_shape=[jax.ShapeDtypeStruct((n, d), jnp.float32),
                   jax.ShapeDtypeStruct((n, 1), jnp.float32)],
    )(agg1, cnt_t, x, Wl1, bl1, Wr1, gamma, beta)

    (agg2,) = seg_sum(h, idx_blocks, rem_blocks)

    out = pl.pallas_call(
        _tc2_body,
        out_shape=jax.ShapeDtypeStruct((n, d), jnp.float32),
    )(agg2, invc, h, Wl2, bl2, Wr2)
    return out


# speedup vs baseline: 11.8757x; 1.0096x over previous
import dataclasses
import functools

import jax
import jax.numpy as jnp
from jax import lax
from jax.experimental import pallas as pl
from jax.experimental.pallas import tpu as pltpu
from jax.experimental.pallas import tpu_sc as plsc

NC = 2
NS = 16
NW = NC * NS
CH = 128


def _chunks(total, step=CH):
    out, o = [], 0
    while o < total:
        sz = min(step, total - o)
        out.append((o, sz))
        o += sz
    return out


def _make_seg_sum(n, d, e, with_cnt):
    epw = e // NW
    nfull = epw // CH
    rem = epw - nfull * CH
    assert epw * NW == e and rem % 16 == 0 and rem < CH
    assert nfull >= 4 and nfull % 2 == 0
    rpt_a = -(-(n // NS) // 8) * 8
    rpt_b = n - (NS - 1) * rpt_a
    assert 0 < rpt_b <= rpt_a

    mesh = plsc.VectorSubcoreMesh(core_axis_name="c", subcore_axis_name="s")
    out_type = [jax.ShapeDtypeStruct((NC, n, d), jnp.float32)]
    scratch = [
        pltpu.VMEM((2, CH), jnp.int32),
        pltpu.VMEM((2, CH), jnp.int32),
        pltpu.VMEM((CH,), jnp.int32),
        pltpu.VMEM((CH,), jnp.int32),
        pltpu.VMEM((CH, d), jnp.float32),
        pltpu.VMEM((CH, d), jnp.float32),
        pltpu.VMEM_SHARED((n, d), jnp.float32),
        pltpu.SemaphoreType.DMA,
        pltpu.SemaphoreType.DMA,
        pltpu.SemaphoreType.DMA,
        pltpu.SemaphoreType.DMA,
        pltpu.SemaphoreType.DMA,
        pltpu.SemaphoreType.DMA,
    ]
    if with_cnt:
        out_type.append(jax.ShapeDtypeStruct((NW, 1, n), jnp.float32))
        scratch.append(pltpu.VMEM((1, n), jnp.float32))
    if rem:
        scratch.append(pltpu.VMEM((2, rem), jnp.int32))

    cp = pltpu.CompilerParams()
    if "needs_layout_passes" in pltpu.CompilerParams.__dataclass_fields__:
        cp = dataclasses.replace(cp, needs_layout_passes=False)

    @functools.partial(pl.kernel, mesh=mesh, out_type=out_type,
                       scratch_types=scratch, compiler_params=cp)
    def seg_sum(x_hbm, idx_hbm, rem_hbm, agg_hbm, *rest):
        rest = list(rest)
        cnt_hbm = rest.pop(0) if with_cnt else None
        (ib0, ib1, didx0, didx1, rows0, rows1, agg_sp,
         isem0, isem1, gsem0, gsem1, ssem0, ssem1) = rest[:13]
        rest = rest[13:]
        if with_cnt:
            cnt_v = rest.pop(0)
        if rem:
            (rbuf,) = rest

        ibufs = (ib0, ib1)
        isems = (isem0, isem1)
        didx_b = (didx0, didx1)
        rows_b = (rows0, rows1)
        gsems = (gsem0, gsem1)
        ssems = (ssem0, ssem1)

        c = lax.axis_index("c")
        s = lax.axis_index("s")
        w = c * NS + s
        z16 = jnp.zeros((16,), jnp.float32)
        z16i = jnp.zeros((16,), jnp.int32)
        one16 = jnp.full((16,), 1.0, jnp.float32)

        def i_start(i, slot):
            pltpu.async_copy(idx_hbm.at[w, i], ibufs[slot], isems[slot])

        def i_wait(i, slot):
            pltpu.make_async_copy(idx_hbm.at[w, i], ibufs[slot],
                                  isems[slot]).wait()

        def g_start(slot):
            pltpu.async_copy(x_hbm.at[ibufs[slot].at[0]], rows_b[slot],
                             gsems[slot])

        def g_wait(slot):
            pltpu.make_async_copy(x_hbm.at[ibufs[slot].at[0]], rows_b[slot],
                                  gsems[slot]).wait()

        def didx_copy(slot):
            for kk in range(CH // 16):
                didx_b[slot][pl.ds(kk * 16, 16)] = (
                    ibufs[slot][1, pl.ds(kk * 16, 16)])

        def s_start(slot):
            pltpu.async_copy(rows_b[slot], agg_sp.at[didx_b[slot]],
                             ssems[slot], add=True)

        def s_wait(slot):
            pltpu.make_async_copy(rows_b[slot], agg_sp.at[didx_b[slot]],
                                  ssems[slot]).wait()

        def cnt_upd(slot):
            if with_cnt:
                for kk in range(CH // 16):
                    idx16 = didx_b[slot][pl.ds(kk * 16, 16)]
                    plsc.addupdate_scatter(cnt_v, [z16i, idx16], one16)

        i_start(0, 0)
        i_start(1, 1)

        @pl.loop(0, CH)
        def _(r):
            for cc in range(d // 16):
                rows0[r, pl.ds(cc * 16, 16)] = z16

        if with_cnt:
            @pl.loop(0, n // 16)
            def _(i):
                cnt_v[0, pl.ds(i * 16, 16)] = z16

        def zero_fill(base, size):
            for (o, sz) in _chunks(size):
                pltpu.sync_copy(rows0.at[pl.ds(0, sz)],
                                agg_sp.at[pl.ds(base + o, sz)])

        def flush(base, size):
            pltpu.sync_copy(agg_sp.at[pl.ds(base, size)],
                            agg_hbm.at[c].at[pl.ds(base, size)])

        base_a = pl.multiple_of(s * rpt_a, 8)

        @pl.when(s < NS - 1)
        def _():
            zero_fill(base_a, rpt_a)

        @pl.when(s == NS - 1)
        def _():
            zero_fill((NS - 1) * rpt_a, rpt_b)

        plsc.subcore_barrier()

        def body(i, slot, start_idx, start_gather, first=False):
            g_wait(slot)
            if start_gather:
                if not first:
                    s_wait(1 - slot)
                i_wait(i + 1, 1 - slot)
                g_start(1 - slot)
            elif not first:
                s_wait(1 - slot)
            didx_copy(slot)
            if start_idx:
                i_start(i + 2, slot)
            s_start(slot)
            cnt_upd(slot)

        i_wait(0, 0)
        g_start(0)
        body(0, 0, True, True, first=True)

        @pl.loop(0, (nfull - 4) // 2)
        def _(j):
            i0 = 2 * j + 1
            body(i0, 1, True, True)
            body(i0 + 1, 0, True, True)

        body(nfull - 3, 1, True, True)
        body(nfull - 2, 0, False, True)
        body(nfull - 1, 1, False, False)
        s_wait(1)

        if rem:
            pltpu.sync_copy(rem_hbm.at[w], rbuf)
            pltpu.sync_copy(x_hbm.at[rbuf.at[0]], rows0.at[pl.ds(0, rem)])
            pltpu.sync_copy(rows0.at[pl.ds(0, rem)],
                            agg_sp.at[rbuf.at[1]], add=True)
            if with_cnt:
                for kk in range(rem // 16):
                    idx16 = rbuf[1, pl.ds(kk * 16, 16)]
                    plsc.addupdate_scatter(cnt_v, [z16i, idx16], one16)

        plsc.subcore_barrier()

        @pl.when(s < NS - 1)
        def _():
            flush(base_a, rpt_a)

        @pl.when(s == NS - 1)
        def _():
            flush((NS - 1) * rpt_a, rpt_b)

        if with_cnt:
            pltpu.sync_copy(cnt_v, cnt_hbm.at[w])

    return seg_sum


def _dot_t(a, w):
    return lax.dot_general(a, w, (((1,), (1,)), ((), ())),
                           preferred_element_type=jnp.float32)


def _tc1_body(aggp, cntp, x, wl, bl, wr, gamma, beta, h_out, invc_out):
    cnt = jnp.sum(cntp[...], axis=1, keepdims=True)
    invc = 1.0 / jnp.maximum(cnt, 1.0)
    mean_agg = (aggp[0] + aggp[1]) * invc
    h = _dot_t(mean_agg, wl[...]) + bl[...][None, :] + _dot_t(x[...], wr[...])
    mu = jnp.mean(h, axis=0, keepdims=True)
    hc = h - mu
    var = jnp.mean(hc * hc, axis=0, keepdims=True)
    hn = hc / jnp.sqrt(var + 1e-5) * gamma[...][None, :] + beta[...][None, :]
    h_out[...] = jnp.maximum(hn, 0.0)
    invc_out[...] = invc


def _tc2_body(aggp, invc, h, wl, bl, wr, out):
    mean_agg = (aggp[0] + aggp[1]) * invc[...]
    out[...] = (_dot_t(mean_agg, wl[...]) + bl[...][None, :]
                + _dot_t(h[...], wr[...]))


def kernel(x, edge_index, Wl1, bl1, Wr1, gamma, beta, Wl2, bl2, Wr2):
    n, d = x.shape
    e = edge_index.shape[1]
    src = edge_index[0].astype(jnp.int32)
    dst = edge_index[1].astype(jnp.int32)

    epw = e // NW
    nfull = epw // CH
    rem = epw - nfull * CH
    srcw = src.reshape(NW, epw)
    dstw = dst.reshape(NW, epw)
    idx_blocks = jnp.stack(
        [srcw[:, :nfull * CH].reshape(NW, nfull, CH),
         dstw[:, :nfull * CH].reshape(NW, nfull, CH)], axis=2)
    rem_blocks = jnp.stack([srcw[:, nfull * CH:], dstw[:, nfull * CH:]],
                           axis=1)

    seg_sum_cnt = _make_seg_sum(n, d, e, with_cnt=True)
    seg_sum = _make_seg_sum(n, d, e, with_cnt=False)

    agg1, cntp = seg_sum_cnt(x, idx_blocks, rem_blocks)
    cnt_t = cntp.reshape(NW, n).T

    h, invc = pl.pallas_call(
        _tc1_body,
        out_shape=[jax.ShapeDtypeStruct((n, d), jnp.float32),
                   jax.ShapeDtypeStruct((n, 1), jnp.float32)],
    )(agg1, cnt_t, x, Wl1, bl1, Wr1, gamma, beta)

    (agg2,) = seg_sum(h, idx_blocks, rem_blocks)

    out = pl.pallas_call(
        _tc2_body,
        out_shape=jax.ShapeDtypeStruct((n, d), jnp.float32),
    )(agg2, invc, h, Wl2, bl2, Wr2)
    return out
